# baseline (speedup 1.0000x reference)
import functools

import jax
import jax.numpy as jnp
from jax.experimental import pallas as pl
from jax.experimental.pallas import tpu as pltpu
from jax.experimental.pallas import tpu_sc as plsc

TOK_BLOCK = 512
PAD_W = 128

SC_NUM_CORES = 2
SC_NUM_SUBCORES = 16


def _vq_argmin_kernel(x_ref, et2_ref, esq_ref, idx_ref):
    x = x_ref[...]
    scores = jax.lax.dot_general(
        x, et2_ref[...], dimension_numbers=(((1,), (0,)), ((), ())),
        preferred_element_type=jnp.float32)
    x_sq = jnp.sum(x * x, axis=1, keepdims=True)
    dist = (x_sq + esq_ref[...]) - 2.0 * scores
    m = jnp.min(dist, axis=1, keepdims=True)
    iota = jax.lax.broadcasted_iota(
        jnp.int32, dist.shape, 1).astype(jnp.float32)
    hit = jnp.where(dist == m, iota, jnp.float32(jnp.inf))
    idx_ref[...] = jnp.min(hit, axis=1).astype(jnp.int32)


def _compute_indices(xf, et2, esq, interpret=False):
    n_tok, c = xf.shape
    n_emb = et2.shape[1]
    grid = n_tok // TOK_BLOCK
    return pl.pallas_call(
        _vq_argmin_kernel,
        grid=(grid,),
        in_specs=[
            pl.BlockSpec((TOK_BLOCK, c), lambda i: (i, 0)),
            pl.BlockSpec((c, n_emb), lambda i: (0, 0)),
            pl.BlockSpec((1, n_emb), lambda i: (0, 0)),
        ],
        out_specs=pl.BlockSpec((TOK_BLOCK,), lambda i: (i,)),
        out_shape=jax.ShapeDtypeStruct((n_tok,), jnp.int32),
        compiler_params=pltpu.CompilerParams(
            dimension_semantics=("parallel",)),
        interpret=interpret,
    )(xf, et2, esq)


def _sc_gather(table, idx, d_out):
    b = idx.shape[0]
    d = table.shape[1]
    nw = SC_NUM_CORES * SC_NUM_SUBCORES
    b_per_w = b // nw
    mesh = plsc.VectorSubcoreMesh(core_axis_name="c", subcore_axis_name="s")

    @functools.partial(
        pl.kernel, mesh=mesh,
        out_type=jax.ShapeDtypeStruct((b, d), jnp.float32),
        scratch_types=[
            pltpu.VMEM((b_per_w,), jnp.int32),
            pltpu.VMEM((b_per_w, d), jnp.float32),
            pltpu.SemaphoreType.DMA,
        ],
    )
    def k(table_hbm, idx_hbm, out_hbm, idx_v, rows_v, sem):
        wid = jax.lax.axis_index("s") * SC_NUM_CORES + jax.lax.axis_index("c")
        base = wid * b_per_w
        pltpu.sync_copy(idx_hbm.at[pl.ds(base, b_per_w)], idx_v)
        pltpu.async_copy(table_hbm.at[idx_v], rows_v, sem).wait()
        pltpu.sync_copy(rows_v, out_hbm.at[pl.ds(base, b_per_w)])

    return k(table, idx)[:, :d_out]


def kernel(x, embedding):
    b, h, w, c = x.shape
    xf = x.reshape(b * h * w, c)
    esq = jnp.sum(embedding ** 2, axis=1)[None, :]
    et = embedding.T
    emb_padded = jnp.pad(embedding, ((0, 0), (0, PAD_W - c)))
    n_tok = xf.shape[0]
    half = n_tok // 2
    idx0 = _compute_indices(xf[:half], et, esq)
    idx1 = _compute_indices(xf[half:], et, esq)
    q0 = _sc_gather(emb_padded, idx0, c)
    q1 = _sc_gather(emb_padded, idx1, c)
    quantized = jnp.concatenate([q0, q1], axis=0)
    return quantized.reshape(b, h, w, c)


# speedup vs baseline: 1.1003x; 1.1003x over previous
import functools

import jax
import jax.numpy as jnp
from jax.experimental import pallas as pl
from jax.experimental.pallas import tpu as pltpu
from jax.experimental.pallas import tpu_sc as plsc

TOK_BLOCK = 512
PAD_W = 128

SC_NUM_CORES = 2
SC_NUM_SUBCORES = 16


def _vq_argmin_kernel(x_ref, et2_ref, esq_ref, idx_ref):
    x = x_ref[...]
    scores = jax.lax.dot_general(
        x, et2_ref[...], dimension_numbers=(((1,), (0,)), ((), ())),
        preferred_element_type=jnp.float32)
    x_sq = jnp.sum(x * x, axis=1, keepdims=True)
    dist = (x_sq + esq_ref[...]) - 2.0 * scores
    m = jnp.min(dist, axis=1, keepdims=True)
    iota = jax.lax.broadcasted_iota(
        jnp.int32, dist.shape, 1).astype(jnp.float32)
    hit = jnp.where(dist == m, iota, jnp.float32(jnp.inf))
    idx_ref[...] = jnp.min(hit, axis=1).astype(jnp.int32)


def _compute_indices(xf, et2, esq, interpret=False):
    n_tok, c = xf.shape
    n_emb = et2.shape[1]
    grid = n_tok // TOK_BLOCK
    return pl.pallas_call(
        _vq_argmin_kernel,
        grid=(grid,),
        in_specs=[
            pl.BlockSpec((TOK_BLOCK, c), lambda i: (i, 0)),
            pl.BlockSpec((c, n_emb), lambda i: (0, 0)),
            pl.BlockSpec((1, n_emb), lambda i: (0, 0)),
        ],
        out_specs=pl.BlockSpec((TOK_BLOCK,), lambda i: (i,)),
        out_shape=jax.ShapeDtypeStruct((n_tok,), jnp.int32),
        compiler_params=pltpu.CompilerParams(
            dimension_semantics=("parallel",)),
        interpret=interpret,
    )(xf, et2, esq)


def _sc_gather(table, idx, d_out):
    b = idx.shape[0]
    d = table.shape[1]
    nw = SC_NUM_CORES * SC_NUM_SUBCORES
    b_per_w = b // nw
    mesh = plsc.VectorSubcoreMesh(core_axis_name="c", subcore_axis_name="s")

    @functools.partial(
        pl.kernel, mesh=mesh,
        out_type=jax.ShapeDtypeStruct((b, d), jnp.float32),
        scratch_types=[
            pltpu.VMEM((b_per_w,), jnp.int32),
            pltpu.VMEM((b_per_w, d), jnp.float32),
            pltpu.SemaphoreType.DMA,
        ],
    )
    def k(table_hbm, idx_hbm, out_hbm, idx_v, rows_v, sem):
        wid = jax.lax.axis_index("s") * SC_NUM_CORES + jax.lax.axis_index("c")
        base = wid * b_per_w
        pltpu.sync_copy(idx_hbm.at[pl.ds(base, b_per_w)], idx_v)
        pltpu.async_copy(table_hbm.at[idx_v], rows_v, sem).wait()
        pltpu.sync_copy(rows_v, out_hbm.at[pl.ds(base, b_per_w)])

    return k(table, idx)[:, :d_out]


def kernel(x, embedding):
    b, h, w, c = x.shape
    xf = x.reshape(b * h * w, c)
    esq = jnp.sum(embedding ** 2, axis=1)[None, :]
    idx = _compute_indices(xf, embedding.T, esq)
    emb_padded = jnp.pad(embedding, ((0, 0), (0, PAD_W - c)))
    quantized = _sc_gather(emb_padded, idx, c)
    return quantized.reshape(b, h, w, c)
